# baseline (speedup 1.0000x reference)
import functools

import jax
import jax.numpy as jnp
from jax import lax
from jax.experimental import pallas as pl
from jax.experimental.pallas import tpu as pltpu
from jax.experimental.pallas import tpu_sc as plsc

_B = 16384
_D = 64
_V = 100000
_NC = 2
_NS = 16
_NW = _NC * _NS
_DPW = _D // _NW
_Q = 4096
_NQ = _B // _Q
_LANES = 16
_UNROLL = 8

_mesh = plsc.VectorSubcoreMesh(core_axis_name="c", subcore_axis_name="s")


@functools.partial(
    pl.kernel,
    out_type=jax.ShapeDtypeStruct((_NW, _LANES), jnp.float32),
    mesh=_mesh,
    compiler_params=pltpu.CompilerParams(needs_layout_passes=False),
    scratch_types=[
        pltpu.VMEM((_V,), jnp.float32),
        pltpu.VMEM((_B,), jnp.int32),
        pltpu.VMEM((2, _Q), jnp.float32),
        pltpu.VMEM((_LANES,), jnp.float32),
        pltpu.SemaphoreType.DMA,
        pltpu.SemaphoreType.DMA,
        pltpu.SemaphoreType.DMA,
    ],
)
def _center_loss_partials(feat_hbm, lab_hbm, cent_hbm, out_hbm,
                          row_v, lab_v, feat_v, acc_v, rsem, lsem, fsem):
    wid = lax.axis_index("s") * _NC + lax.axis_index("c")

    lcopy = pltpu.async_copy(lab_hbm, lab_v, lsem)
    rcopy = pltpu.async_copy(cent_hbm.at[wid * _DPW], row_v, rsem)
    lcopy.wait()

    accs = [jnp.zeros((_LANES,), jnp.float32) for _ in range(4)]
    for k in range(_DPW):
        d = wid * _DPW + k
        fcopy = pltpu.async_copy(feat_hbm.at[d, pl.ds(0, _Q)],
                                 feat_v.at[0], fsem)
        rcopy.wait()
        for q in range(_NQ):
            fcopy.wait()
            if q + 1 < _NQ:
                fcopy = pltpu.async_copy(
                    feat_hbm.at[d, pl.ds((q + 1) * _Q, _Q)],
                    feat_v.at[(q + 1) % 2], fsem)
            elif k + 1 < _DPW:
                pass

            def step(i, a, _q=q):
                a = list(a)
                for u in range(_UNROLL):
                    off = i * _UNROLL * _LANES + u * _LANES
                    idx = lab_v[pl.ds(_q * _Q + off, _LANES)]
                    g = plsc.load_gather(row_v, [idx])
                    f = feat_v[_q % 2, pl.ds(off, _LANES)]
                    e = f - g
                    a[u % 4] = a[u % 4] + e * e
                return tuple(a)

            accs = list(lax.fori_loop(0, _Q // (_LANES * _UNROLL), step,
                                      tuple(accs)))
        if k + 1 < _DPW:
            rcopy = pltpu.async_copy(cent_hbm.at[d + 1], row_v, rsem)

    acc_v[...] = ((accs[0] + accs[1]) + (accs[2] + accs[3])) * (0.5 / _B)
    pltpu.sync_copy(acc_v, out_hbm.at[wid])


def kernel(features, labels, centers):
    partials = _center_loss_partials(features.T, labels.astype(jnp.int32),
                                     centers.T)
    return jnp.sum(partials)


# speedup vs baseline: 1.0021x; 1.0021x over previous
import functools

import jax
import jax.numpy as jnp
from jax import lax
from jax.experimental import pallas as pl
from jax.experimental.pallas import tpu as pltpu
from jax.experimental.pallas import tpu_sc as plsc

_B = 16384
_D = 64
_V = 100000
_NC = 2
_NS = 16
_NW = _NC * _NS
_DPW = _D // _NW
_Q = 4096
_NQ = _B // _Q
_LANES = 16
_UNROLL = 8

_mesh = plsc.VectorSubcoreMesh(core_axis_name="c", subcore_axis_name="s")


@functools.partial(
    pl.kernel,
    out_type=jax.ShapeDtypeStruct((_NW, _LANES), jnp.float32),
    mesh=_mesh,
    compiler_params=pltpu.CompilerParams(needs_layout_passes=False),
    scratch_types=[
        pltpu.VMEM((_V,), jnp.float32),
        pltpu.VMEM((_B,), jnp.int32),
        pltpu.VMEM((2, _Q), jnp.float32),
        pltpu.VMEM((_LANES,), jnp.float32),
        pltpu.SemaphoreType.DMA,
        pltpu.SemaphoreType.DMA,
        pltpu.SemaphoreType.DMA,
    ],
)
def _center_loss_partials(feat_hbm, lab_hbm, cent_hbm, out_hbm,
                          row_v, lab_v, feat_v, acc_v, rsem, lsem, fsem):
    wid = lax.axis_index("s") * _NC + lax.axis_index("c")

    lcopy = pltpu.async_copy(lab_hbm, lab_v, lsem)
    rcopy = pltpu.async_copy(cent_hbm.at[wid * _DPW], row_v, rsem)
    lcopy.wait()

    accs = [jnp.zeros((_LANES,), jnp.float32) for _ in range(4)]
    for k in range(_DPW):
        d = wid * _DPW + k
        fcopy = pltpu.async_copy(feat_hbm.at[d, pl.ds(0, _Q)],
                                 feat_v.at[0], fsem)
        rcopy.wait()
        for q in range(_NQ):
            fcopy.wait()
            if q + 1 < _NQ:
                fcopy = pltpu.async_copy(
                    feat_hbm.at[d, pl.ds((q + 1) * _Q, _Q)],
                    feat_v.at[(q + 1) % 2], fsem)
            elif k + 1 < _DPW:
                pass

            def step(i, a, _q=q):
                off = i * _LANES
                idx = lab_v[pl.ds(_q * _Q + off, _LANES)]
                g = plsc.load_gather(row_v, [idx])
                f = feat_v[_q % 2, pl.ds(off, _LANES)]
                e = f - g
                a = list(a)
                a[0] = a[0] + e * e
                return tuple(a[1:] + a[:1])

            accs = list(plsc.parallel_loop(0, _Q // _LANES, 1,
                                           unroll=_UNROLL,
                                           carry=tuple(accs))(step))
        if k + 1 < _DPW:
            rcopy = pltpu.async_copy(cent_hbm.at[d + 1], row_v, rsem)

    acc_v[...] = ((accs[0] + accs[1]) + (accs[2] + accs[3])) * (0.5 / _B)
    pltpu.sync_copy(acc_v, out_hbm.at[wid])


def kernel(features, labels, centers):
    partials = _center_loss_partials(features.T, labels.astype(jnp.int32),
                                     centers.T)
    return jnp.sum(partials)
